# Initial kernel scaffold; baseline (speedup 1.0000x reference)
import jax
import jax.numpy as jnp
from jax.experimental import pallas as pl


def kernel(mouse_cat, scroll, buttons, keys, yaw_pitch, gui, hotbar, mouse_table, scroll_table, hotbar_table, slot_table, buttons_W, buttons_b, keys_W, keys_b, yawgui_W, yawgui_b):
    raise NotImplementedError("write your pallas kernel here")


# speedup vs baseline: 1.2392x; 1.2392x over previous
import functools

import jax
import jax.numpy as jnp
from jax import lax
from jax.experimental import pallas as pl
from jax.experimental.pallas import tpu as pltpu
from jax.experimental.pallas import tpu_sc as plsc

B, T, D = 32, 256, 1024
BT = B * T
NSLOT = 4


def _sc_gather_slot0(table_biased, idx_flat):
    info = plsc.get_sparse_core_info()
    nw = info.num_cores * info.num_subcores
    per_w = BT // nw
    chunk = 64
    n_chunks = per_w // chunk

    mesh = plsc.VectorSubcoreMesh(core_axis_name="c", subcore_axis_name="s")

    @functools.partial(
        pl.kernel,
        mesh=mesh,
        out_type=jax.ShapeDtypeStruct((BT, NSLOT * D), jnp.float32),
        scratch_types=[
            pltpu.VMEM((chunk,), jnp.int32),
            pltpu.VMEM((chunk, D), jnp.float32),
            pltpu.SemaphoreType.DMA,
        ],
    )
    def k(table_hbm, idx_hbm, out_hbm, idx_v, rows_v, sem):
        wid = lax.axis_index("s") * info.num_cores + lax.axis_index("c")
        base = wid * per_w

        def body(c, carry):
            off = base + c * chunk
            pltpu.sync_copy(idx_hbm.at[pl.ds(off, chunk)], idx_v)
            pltpu.async_copy(table_hbm.at[idx_v], rows_v, sem).wait()
            pltpu.sync_copy(rows_v, out_hbm.at[pl.ds(off, chunk), pl.ds(0, D)])
            return carry

        lax.fori_loop(0, n_chunks, body, 0)

    return k(table_biased, idx_flat)


def _tc_dense(tokens0, scroll_r, hotbar_r, buttons, keys, yaw_pitch, gui,
              scroll_table, buttons_W, keys_W, w_yp, w_gui, hotbar_table,
              bias3):
    def body(alias_ref, scroll_ref, hotbar_ref, btn_ref, keys_ref, yp_ref,
             gui_ref, st_ref, bw_ref, kw_ref, wyp_ref, wgui_ref, ht_ref,
             bias_ref, out_ref):
        s = pl.program_id(1)
        f32 = jnp.float32

        @pl.when(s == 0)
        def _slot1():
            sc = scroll_ref[0, 0, :][:, None]
            oh = (sc == lax.broadcasted_iota(jnp.int32, (T, 3), 1)).astype(f32)
            out_ref[...] = (
                jnp.dot(oh, st_ref[...], preferred_element_type=f32)
                + jnp.dot(btn_ref[0], bw_ref[...], preferred_element_type=f32)
                + bias_ref[0]
            )

        @pl.when(s == 1)
        def _slot2():
            out_ref[...] = (
                jnp.dot(keys_ref[0], kw_ref[...], preferred_element_type=f32)
                + bias_ref[0]
            )

        @pl.when(s == 2)
        def _slot3():
            hb = hotbar_ref[0, 0, :][:, None]
            oh = (hb == lax.broadcasted_iota(jnp.int32, (T, 9), 1)).astype(f32)
            out_ref[...] = (
                jnp.dot(yp_ref[0], wyp_ref[...], preferred_element_type=f32)
                + jnp.dot(gui_ref[0], wgui_ref[...], preferred_element_type=f32)
                + jnp.dot(oh, ht_ref[...], preferred_element_type=f32)
                + bias_ref[0]
            )

    full = lambda shape: pl.BlockSpec(shape, lambda b, s: (0,) * len(shape))
    per_b = lambda shape: pl.BlockSpec(shape, lambda b, s: (b,) + (0,) * (len(shape) - 1))

    return pl.pallas_call(
        body,
        grid=(B, 3),
        in_specs=[
            pl.BlockSpec(memory_space=pl.ANY),
            per_b((1, 1, T)),
            per_b((1, 1, T)),
            per_b((1, T, 3)),
            per_b((1, T, 23)),
            per_b((1, T, 2)),
            per_b((1, T, 2)),
            full((3, D)),
            full((3, D)),
            full((23, D)),
            full((2, D)),
            full((2, D)),
            full((9, D)),
            pl.BlockSpec((1, 1, D), lambda b, s: (s, 0, 0)),
        ],
        out_specs=pl.BlockSpec((T, D), lambda b, s: (b, s + 1)),
        out_shape=jax.ShapeDtypeStruct((BT, NSLOT * D), jnp.float32),
        input_output_aliases={0: 0},
    )(tokens0, scroll_r, hotbar_r, buttons, keys, yaw_pitch, gui,
      scroll_table, buttons_W, keys_W, w_yp, w_gui, hotbar_table, bias3)


def kernel(mouse_cat, scroll, buttons, keys, yaw_pitch, gui, hotbar,
           mouse_table, scroll_table, hotbar_table, slot_table,
           buttons_W, buttons_b, keys_W, keys_b, yawgui_W, yawgui_b):
    table_biased = mouse_table + slot_table[0][None, :]
    bias3 = jnp.stack([
        slot_table[1] + buttons_b,
        slot_table[2] + keys_b,
        slot_table[3] + yawgui_b,
    ])[:, None, :]
    w_yp = yawgui_W[:2]
    w_gui = yawgui_W[2:]

    idx_flat = mouse_cat.reshape(BT).astype(jnp.int32)
    scroll_r = scroll.reshape(B, 1, T).astype(jnp.int32)
    hotbar_r = hotbar.reshape(B, 1, T).astype(jnp.int32)

    tokens0 = _sc_gather_slot0(table_biased, idx_flat)
    tokens = _tc_dense(tokens0, scroll_r, hotbar_r, buttons, keys,
                       yaw_pitch, gui, scroll_table, buttons_W, keys_W,
                       w_yp, w_gui, hotbar_table, bias3)
    return tokens.reshape(B, T, NSLOT, D)


# speedup vs baseline: 1.4072x; 1.1356x over previous
import functools

import jax
import jax.numpy as jnp
from jax import lax
from jax.experimental import pallas as pl
from jax.experimental.pallas import tpu as pltpu
from jax.experimental.pallas import tpu_sc as plsc

B, T, D = 32, 256, 1024
BT = B * T
NSLOT = 4


def _sc_gather_slot0(table_biased, idx_flat):
    info = plsc.get_sparse_core_info()
    nw = info.num_cores * info.num_subcores
    per_w = BT // nw
    chunk = 64
    n_chunks = per_w // chunk

    mesh = plsc.VectorSubcoreMesh(core_axis_name="c", subcore_axis_name="s")

    @functools.partial(
        pl.kernel,
        mesh=mesh,
        out_type=jax.ShapeDtypeStruct((BT, NSLOT * D), jnp.float32),
        scratch_types=[
            pltpu.VMEM((chunk,), jnp.int32),
            pltpu.VMEM((chunk, D), jnp.float32),
            pltpu.SemaphoreType.DMA,
        ],
    )
    def k(table_hbm, idx_hbm, out_hbm, idx_v, rows_v, sem):
        wid = lax.axis_index("s") * info.num_cores + lax.axis_index("c")
        base = wid * per_w

        def body(c, carry):
            off = base + c * chunk
            pltpu.sync_copy(idx_hbm.at[pl.ds(off, chunk)], idx_v)
            pltpu.async_copy(table_hbm.at[idx_v], rows_v, sem).wait()
            pltpu.sync_copy(rows_v, out_hbm.at[pl.ds(off, chunk), pl.ds(0, D)])
            return carry

        lax.fori_loop(0, n_chunks, body, 0)

    return k(table_biased, idx_flat)


BR = 1024


def _tc_dense(tokens0, scroll_r, hotbar_r, buttons, keys, yaw_pitch, gui,
              scroll_table, buttons_W, keys_W, w_yp, w_gui, hotbar_table,
              bias3):
    nb = BT // BR

    def body(alias_ref, scroll_ref, hotbar_ref, btn_ref, keys_ref, yp_ref,
             gui_ref, st_ref, bw_ref, kw_ref, wyp_ref, wgui_ref, ht_ref,
             bias_ref, out_ref):
        s = pl.program_id(1)
        f32 = jnp.float32

        @pl.when(s == 0)
        def _slot1():
            sc = scroll_ref[0, 0, :][:, None]
            oh = (sc == lax.broadcasted_iota(jnp.int32, (BR, 3), 1)).astype(f32)
            out_ref[...] = (
                jnp.dot(oh, st_ref[...], preferred_element_type=f32)
                + jnp.dot(btn_ref[...], bw_ref[...], preferred_element_type=f32)
                + bias_ref[0]
            )

        @pl.when(s == 1)
        def _slot2():
            out_ref[...] = (
                jnp.dot(keys_ref[...], kw_ref[...], preferred_element_type=f32)
                + bias_ref[0]
            )

        @pl.when(s == 2)
        def _slot3():
            hb = hotbar_ref[0, 0, :][:, None]
            oh = (hb == lax.broadcasted_iota(jnp.int32, (BR, 9), 1)).astype(f32)
            out_ref[...] = (
                jnp.dot(yp_ref[...], wyp_ref[...], preferred_element_type=f32)
                + jnp.dot(gui_ref[...], wgui_ref[...], preferred_element_type=f32)
                + jnp.dot(oh, ht_ref[...], preferred_element_type=f32)
                + bias_ref[0]
            )

    full = lambda shape: pl.BlockSpec(shape, lambda b, s: (0,) * len(shape))
    per_b = lambda shape: pl.BlockSpec(shape, lambda b, s: (b,) + (0,) * (len(shape) - 1))

    return pl.pallas_call(
        body,
        grid=(nb, 3),
        in_specs=[
            pl.BlockSpec(memory_space=pl.ANY),
            per_b((1, 1, BR)),
            per_b((1, 1, BR)),
            per_b((BR, 3)),
            per_b((BR, 23)),
            per_b((BR, 2)),
            per_b((BR, 2)),
            full((3, D)),
            full((3, D)),
            full((23, D)),
            full((2, D)),
            full((2, D)),
            full((9, D)),
            pl.BlockSpec((1, 1, D), lambda b, s: (s, 0, 0)),
        ],
        out_specs=pl.BlockSpec((BR, D), lambda b, s: (b, s + 1)),
        out_shape=jax.ShapeDtypeStruct((BT, NSLOT * D), jnp.float32),
        input_output_aliases={0: 0},
    )(tokens0, scroll_r, hotbar_r, buttons, keys, yaw_pitch, gui,
      scroll_table, buttons_W, keys_W, w_yp, w_gui, hotbar_table, bias3)


def kernel(mouse_cat, scroll, buttons, keys, yaw_pitch, gui, hotbar,
           mouse_table, scroll_table, hotbar_table, slot_table,
           buttons_W, buttons_b, keys_W, keys_b, yawgui_W, yawgui_b):
    table_biased = mouse_table + slot_table[0][None, :]
    bias3 = jnp.stack([
        slot_table[1] + buttons_b,
        slot_table[2] + keys_b,
        slot_table[3] + yawgui_b,
    ])[:, None, :]
    w_yp = yawgui_W[:2]
    w_gui = yawgui_W[2:]

    idx_flat = mouse_cat.reshape(BT).astype(jnp.int32)
    scroll_r = scroll.reshape(BT // BR, 1, BR).astype(jnp.int32)
    hotbar_r = hotbar.reshape(BT // BR, 1, BR).astype(jnp.int32)

    tokens0 = _sc_gather_slot0(table_biased, idx_flat)
    tokens = _tc_dense(tokens0, scroll_r, hotbar_r,
                       buttons.reshape(BT, 3), keys.reshape(BT, 23),
                       yaw_pitch.reshape(BT, 2), gui.reshape(BT, 2),
                       scroll_table, buttons_W, keys_W,
                       w_yp, w_gui, hotbar_table, bias3)
    return tokens.reshape(B, T, NSLOT, D)


# speedup vs baseline: 1.4507x; 1.0309x over previous
import functools

import jax
import jax.numpy as jnp
from jax import lax
from jax.experimental import pallas as pl
from jax.experimental.pallas import tpu as pltpu
from jax.experimental.pallas import tpu_sc as plsc

B, T, D = 32, 256, 1024
BT = B * T
NSLOT = 4


def _sc_gather_slot0(table_biased, idx_flat):
    info = plsc.get_sparse_core_info()
    nw = info.num_cores * info.num_subcores
    per_w = BT // nw
    chunk = 64
    n_chunks = per_w // chunk

    mesh = plsc.VectorSubcoreMesh(core_axis_name="c", subcore_axis_name="s")

    @functools.partial(
        pl.kernel,
        mesh=mesh,
        out_type=jax.ShapeDtypeStruct((BT, NSLOT * D), jnp.float32),
        scratch_types=[
            pltpu.VMEM((chunk,), jnp.int32),
            pltpu.VMEM((chunk, D), jnp.float32),
            pltpu.SemaphoreType.DMA,
        ],
    )
    def k(table_hbm, idx_hbm, out_hbm, idx_v, rows_v, sem):
        wid = lax.axis_index("s") * info.num_cores + lax.axis_index("c")
        base = wid * per_w

        def body(c, carry):
            off = base + c * chunk
            pltpu.sync_copy(idx_hbm.at[pl.ds(off, chunk)], idx_v)
            pltpu.async_copy(table_hbm.at[idx_v], rows_v, sem).wait()
            pltpu.sync_copy(rows_v, out_hbm.at[pl.ds(off, chunk), pl.ds(0, D)])
            return carry

        lax.fori_loop(0, n_chunks, body, 0)

    return k(table_biased, idx_flat)


BR = 1024


def _tc_dense(tokens0, scroll_r, hotbar_r, buttons, keys, yaw_pitch, gui,
              scroll_table, buttons_W, keys_W, w_yp, w_gui, hotbar_table,
              bias3):
    nb = BT // BR

    def body(alias_ref, scroll_ref, hotbar_ref, btn_ref, keys_ref, yp_ref,
             gui_ref, st_ref, bw_ref, kw_ref, wyp_ref, wgui_ref, ht_ref,
             bias_ref, out_ref):
        f32 = jnp.float32
        sc = scroll_ref[0, 0, :][:, None]
        oh_s = (sc == lax.broadcasted_iota(jnp.int32, (BR, 3), 1)).astype(f32)
        out_ref[:, :D] = (
            jnp.dot(oh_s, st_ref[...], preferred_element_type=f32)
            + jnp.dot(btn_ref[...], bw_ref[...], preferred_element_type=f32)
            + bias_ref[0, 0]
        )
        out_ref[:, D:2 * D] = (
            jnp.dot(keys_ref[...], kw_ref[...], preferred_element_type=f32)
            + bias_ref[1, 0]
        )
        hb = hotbar_ref[0, 0, :][:, None]
        oh_h = (hb == lax.broadcasted_iota(jnp.int32, (BR, 9), 1)).astype(f32)
        out_ref[:, 2 * D:] = (
            jnp.dot(yp_ref[...], wyp_ref[...], preferred_element_type=f32)
            + jnp.dot(gui_ref[...], wgui_ref[...], preferred_element_type=f32)
            + jnp.dot(oh_h, ht_ref[...], preferred_element_type=f32)
            + bias_ref[2, 0]
        )

    full = lambda shape: pl.BlockSpec(shape, lambda b: (0,) * len(shape))
    per_b = lambda shape: pl.BlockSpec(shape, lambda b: (b,) + (0,) * (len(shape) - 1))

    return pl.pallas_call(
        body,
        grid=(nb,),
        in_specs=[
            pl.BlockSpec(memory_space=pl.ANY),
            per_b((1, 1, BR)),
            per_b((1, 1, BR)),
            per_b((BR, 3)),
            per_b((BR, 23)),
            per_b((BR, 2)),
            per_b((BR, 2)),
            full((3, D)),
            full((3, D)),
            full((23, D)),
            full((2, D)),
            full((2, D)),
            full((9, D)),
            full((3, 1, D)),
        ],
        out_specs=pl.BlockSpec((pl.Element(BR), pl.Element(3 * D)),
                               lambda b: (b * BR, D)),
        out_shape=jax.ShapeDtypeStruct((BT, NSLOT * D), jnp.float32),
        input_output_aliases={0: 0},
    )(tokens0, scroll_r, hotbar_r, buttons, keys, yaw_pitch, gui,
      scroll_table, buttons_W, keys_W, w_yp, w_gui, hotbar_table, bias3)


def kernel(mouse_cat, scroll, buttons, keys, yaw_pitch, gui, hotbar,
           mouse_table, scroll_table, hotbar_table, slot_table,
           buttons_W, buttons_b, keys_W, keys_b, yawgui_W, yawgui_b):
    table_biased = mouse_table + slot_table[0][None, :]
    bias3 = jnp.stack([
        slot_table[1] + buttons_b,
        slot_table[2] + keys_b,
        slot_table[3] + yawgui_b,
    ])[:, None, :]
    w_yp = yawgui_W[:2]
    w_gui = yawgui_W[2:]

    idx_flat = mouse_cat.reshape(BT).astype(jnp.int32)
    scroll_r = scroll.reshape(BT // BR, 1, BR).astype(jnp.int32)
    hotbar_r = hotbar.reshape(BT // BR, 1, BR).astype(jnp.int32)

    tokens0 = _sc_gather_slot0(table_biased, idx_flat)
    tokens = _tc_dense(tokens0, scroll_r, hotbar_r,
                       buttons.reshape(BT, 3), keys.reshape(BT, 23),
                       yaw_pitch.reshape(BT, 2), gui.reshape(BT, 2),
                       scroll_table, buttons_W, keys_W,
                       w_yp, w_gui, hotbar_table, bias3)
    return tokens.reshape(B, T, NSLOT, D)
